# baseline (speedup 1.0000x reference)
import functools

import jax
import jax.numpy as jnp
import numpy as np
from jax import lax
from jax.experimental import pallas as pl
from jax.experimental.pallas import tpu as pltpu
from jax.experimental.pallas import tpu_sc as plsc

N_NODES = 10000
NP = 10112
CHUNK = 64
NCH0 = 280
NCH1 = 36
TOTCH = (NCH0 + NCH1) * 16
EPAD = TOTCH * CHUNK
RPT = NP // 16


def _sc_agg(D, DBF):
  mesh = plsc.VectorSubcoreMesh(core_axis_name="c", subcore_axis_name="s")

  @functools.partial(
      pl.kernel,
      out_type=jax.ShapeDtypeStruct((2, NP, D), jnp.float32),
      mesh=mesh,
      scratch_types=[
          pltpu.VMEM((6, 2, CHUNK), jnp.int32),
          pltpu.VMEM((3, CHUNK, DBF), jnp.bfloat16),
          pltpu.VMEM((2, CHUNK, D), jnp.float32),
          pltpu.VMEM_SHARED((NP, D), jnp.float32),
          pltpu.SemaphoreType.DMA((3,)),
          pltpu.SemaphoreType.DMA((2,)),
          pltpu.SemaphoreType.DMA((6,)),
      ],
      compiler_params=pltpu.CompilerParams(use_tc_tiling_on_sc=False,
                                           needs_layout_passes=False),
  )
  def k(h_hbm, edges_hbm, zero_hbm, out_hbm, idx_v, rbf_v, rows_v, acc_s,
        gsem, ssem, isem):
    cid = lax.axis_index("c")
    sid = lax.axis_index("s")
    start = lax.select(cid == 0, sid * NCH0, 16 * NCH0 + sid * NCH1)
    n_my = lax.select(cid == 0, NCH0, NCH1)

    pltpu.sync_copy(zero_hbm, rows_v.at[0])
    def zbody(j, carry):
      pltpu.sync_copy(rows_v.at[0], acc_s.at[pl.ds(sid * RPT + j * CHUNK, CHUNK)])
      return carry
    lax.fori_loop(0, RPT // CHUNK, zbody, 0)
    pltpu.sync_copy(rows_v.at[0, pl.ds(0, RPT - (RPT // CHUNK) * CHUNK)],
                    acc_s.at[pl.ds(sid * RPT + (RPT // CHUNK) * CHUNK,
                                   RPT - (RPT // CHUNK) * CHUNK)])
    plsc.subcore_barrier()

    pltpu.sync_copy(edges_hbm.at[start], idx_v.at[0])
    pltpu.sync_copy(edges_hbm.at[start + 1], idx_v.at[1])
    pltpu.sync_copy(edges_hbm.at[start + 2], idx_v.at[2])
    pltpu.async_copy(h_hbm.at[idx_v.at[0, 0]], rbf_v.at[0], gsem.at[0])
    pltpu.async_copy(h_hbm.at[idx_v.at[1, 0]], rbf_v.at[1], gsem.at[1])

    nfull = min(DBF, D) // 32
    extra = D > nfull * 32

    def body(c, carry):
      g = lax.rem(c, 3)
      g2 = lax.rem(c + 2, 3)
      fb = lax.rem(c, 2)
      j = lax.rem(c, 6)
      j2 = lax.rem(c + 2, 6)
      j3 = lax.rem(c + 3, 6)

      @pl.when(c + 2 < n_my)
      def _():
        @pl.when(c >= 1)
        def _():
          pltpu.make_async_copy(edges_hbm.at[start + c + 2], idx_v.at[j2],
                                isem.at[j2]).wait()
        pltpu.async_copy(h_hbm.at[idx_v.at[j2, 0]], rbf_v.at[g2],
                         gsem.at[g2])

      @pl.when(c + 3 < n_my)
      def _():
        pltpu.async_copy(edges_hbm.at[start + c + 3], idx_v.at[j3],
                         isem.at[j3])

      pltpu.make_async_copy(h_hbm.at[idx_v.at[j, 0]], rbf_v.at[g],
                            gsem.at[g]).wait()
      @pl.when(c >= 2)
      def _():
        pltpu.make_async_copy(rows_v.at[fb], acc_s.at[idx_v.at[j, 1]],
                              ssem.at[fb]).wait()

      def conv(i, carry2):
        for gg in range(nfull):
          vi = plsc.bitcast(rbf_v[g, i, pl.ds(gg * 32, 32)], jnp.int32)
          rows_v[fb, i, pl.ds(gg * 32, 16)] = plsc.bitcast(
              vi << 16, jnp.float32)
          rows_v[fb, i, pl.ds(gg * 32 + 16, 16)] = plsc.bitcast(
              vi & jnp.int32(-65536), jnp.float32)
        if extra:
          vi = plsc.bitcast(rbf_v[g, i, pl.ds(nfull * 32, 32)], jnp.int32)
          rows_v[fb, i, pl.ds(nfull * 32, 16)] = plsc.bitcast(
              vi << 16, jnp.float32)
        return carry2

      lax.fori_loop(0, CHUNK, conv, 0, unroll=2)

      pltpu.async_copy(rows_v.at[fb], acc_s.at[idx_v.at[j, 1]], ssem.at[fb],
                       add=True)
      return carry

    lax.fori_loop(0, n_my, body, 0)

    def drain(kk, carry):
      pltpu.make_async_copy(rows_v.at[lax.rem(kk, 2)],
                            acc_s.at[idx_v.at[lax.rem(kk, 6), 1]],
                            ssem.at[lax.rem(kk, 2)]).wait()
      return carry

    lax.fori_loop(n_my - 2, n_my, drain, 0)
    plsc.subcore_barrier()

    def wbody(j, carry):
      r = sid * RPT + j * CHUNK
      pltpu.sync_copy(acc_s.at[pl.ds(r, CHUNK)], rows_v.at[0])
      pltpu.sync_copy(rows_v.at[0], out_hbm.at[cid, pl.ds(r, CHUNK)])
      return carry

    lax.fori_loop(0, RPT // CHUNK, wbody, 0)
    _tail = RPT - (RPT // CHUNK) * CHUNK
    r = sid * RPT + (RPT // CHUNK) * CHUNK
    pltpu.sync_copy(acc_s.at[pl.ds(r, _tail)], rows_v.at[0, pl.ds(0, _tail)])
    pltpu.sync_copy(rows_v.at[0, pl.ds(0, _tail)],
                    out_hbm.at[cid, pl.ds(r, _tail)])

  return k


_R = 1264


def _tc1(p, w, b):
  def body(p_ref, w_ref, b_ref, h_ref, rc_ref):
    s = p_ref[0] + p_ref[1]
    rc = 1.0 / jnp.maximum(s[:, 128:129], 1.0)
    x = s[:, :128] * rc
    y = lax.dot_general(x, w_ref[...], (((1,), (1,)), ((), ())),
                        preferred_element_type=jnp.float32) + b_ref[...]
    h_ref[...] = jnp.where(y > 0, y, jnp.exp(jnp.minimum(y, 0.0)) - 1.0
                           ).astype(jnp.bfloat16)
    rc_ref[...] = rc

  return pl.pallas_call(
      body,
      grid=(NP // _R,),
      in_specs=[
          pl.BlockSpec((2, _R, 144), lambda i: (0, i, 0)),
          pl.BlockSpec((128, 128), lambda i: (0, 0)),
          pl.BlockSpec((1, 128), lambda i: (0, 0)),
      ],
      out_specs=[
          pl.BlockSpec((_R, 128), lambda i: (i, 0)),
          pl.BlockSpec((_R, 1), lambda i: (i, 0)),
      ],
      out_shape=[
          jax.ShapeDtypeStruct((NP, 128), jnp.bfloat16),
          jax.ShapeDtypeStruct((NP, 1), jnp.float32),
      ],
  )(p, w, b)


def _tc2(p, rc, w, b):
  def body(p_ref, rc_ref, w_ref, b_ref, o_ref):
    x = (p_ref[0] + p_ref[1]) * rc_ref[...]
    y = lax.dot_general(x, w_ref[...], (((1,), (1,)), ((), ())),
                        preferred_element_type=jnp.float32) + b_ref[...]
    o_ref[...] = jnp.where(y > 0, y, jnp.exp(jnp.minimum(y, 0.0)) - 1.0)

  return pl.pallas_call(
      body,
      grid=(NP // _R,),
      in_specs=[
          pl.BlockSpec((2, _R, 128), lambda i: (0, i, 0)),
          pl.BlockSpec((_R, 1), lambda i: (i, 0)),
          pl.BlockSpec((128, 128), lambda i: (0, 0)),
          pl.BlockSpec((1, 128), lambda i: (0, 0)),
      ],
      out_specs=pl.BlockSpec((_R, 128), lambda i: (i, 0)),
      out_shape=jax.ShapeDtypeStruct((NP, 128), jnp.float32),
  )(p, rc, w, b)


def kernel(h, edge_index, W1, b1, W2, b2):
  E = edge_index.shape[1]
  pad = EPAD - E
  src = jnp.concatenate([edge_index[0].astype(jnp.int32),
                         jnp.zeros((pad,), jnp.int32)]).reshape(TOTCH, CHUNK)
  dst = jnp.concatenate([edge_index[1].astype(jnp.int32),
                         jnp.full((pad,), N_NODES, jnp.int32)]
                        ).reshape(TOTCH, CHUNK)
  edges = jnp.stack([src, dst], axis=1)
  h_aug = jnp.concatenate(
      [h.astype(jnp.bfloat16), jnp.ones((N_NODES, 1), jnp.bfloat16),
       jnp.zeros((N_NODES, 31), jnp.bfloat16)], axis=1)

  sig = np.empty(128, np.int64)
  for c in range(128):
    g, j = divmod(c, 32)
    sig[c] = g * 32 + (j % 2) * 16 + j // 2
  siginv = np.argsort(sig)
  W1p = W1[:, siginv]
  W2p = W2[:, siginv]

  z144 = jnp.zeros((CHUNK, 144), jnp.float32)
  z128 = jnp.zeros((CHUNK, 128), jnp.float32)

  p1 = _sc_agg(144, 160)(h_aug, edges, z144)
  h1, rc = _tc1(p1, W1p, b1.reshape(1, 128))
  p2 = _sc_agg(128, 128)(h1, edges, z128)
  out = _tc2(p2, rc, W2p, b2.reshape(1, 128))
  return out[:N_NODES]


# speedup vs baseline: 1.7125x; 1.7125x over previous
import functools

import jax
import jax.numpy as jnp
from jax import lax
from jax.experimental import pallas as pl
from jax.experimental.pallas import tpu as pltpu
from jax.experimental.pallas import tpu_sc as plsc

N_NODES = 10000
NP = 10112
CHUNK = 64
NCH0 = 296
NCH1 = 20
TOTCH = (NCH0 + NCH1) * 16
EPAD = TOTCH * CHUNK
RPT = NP // 16


def _sc_agg(D):
  mesh = plsc.VectorSubcoreMesh(core_axis_name="c", subcore_axis_name="s")

  @functools.partial(
      pl.kernel,
      out_type=jax.ShapeDtypeStruct((2, NP, D), jnp.float32),
      mesh=mesh,
      scratch_types=[
          pltpu.VMEM((6, 2, CHUNK), jnp.int32),
          pltpu.VMEM((4, CHUNK, D), jnp.float32),
          pltpu.VMEM_SHARED((NP, D), jnp.float32),
          pltpu.SemaphoreType.DMA((4,)),
          pltpu.SemaphoreType.DMA((4,)),
          pltpu.SemaphoreType.DMA((6,)),
      ],
      compiler_params=pltpu.CompilerParams(use_tc_tiling_on_sc=False),
  )
  def k(h_hbm, edges_hbm, zero_hbm, out_hbm, idx_v, rows_v, acc_s,
        gsem, ssem, isem):
    cid = lax.axis_index("c")
    sid = lax.axis_index("s")
    start = lax.select(cid == 0, sid * NCH0, 16 * NCH0 + sid * NCH1)
    n_my = lax.select(cid == 0, NCH0, NCH1)

    pltpu.sync_copy(zero_hbm, rows_v.at[0])
    def zbody(j, carry):
      pltpu.sync_copy(rows_v.at[0], acc_s.at[pl.ds(sid * RPT + j * CHUNK, CHUNK)])
      return carry
    lax.fori_loop(0, RPT // CHUNK, zbody, 0)
    pltpu.sync_copy(rows_v.at[0, pl.ds(0, RPT - (RPT // CHUNK) * CHUNK)],
                    acc_s.at[pl.ds(sid * RPT + (RPT // CHUNK) * CHUNK,
                                   RPT - (RPT // CHUNK) * CHUNK)])
    plsc.subcore_barrier()

    pltpu.sync_copy(edges_hbm.at[start], idx_v.at[0])
    pltpu.sync_copy(edges_hbm.at[start + 1], idx_v.at[1])
    pltpu.sync_copy(edges_hbm.at[start + 2], idx_v.at[2])
    pltpu.async_copy(h_hbm.at[idx_v.at[0, 0]], rows_v.at[0], gsem.at[0])
    pltpu.async_copy(h_hbm.at[idx_v.at[1, 0]], rows_v.at[1], gsem.at[1])

    def body(c, carry):
      b = lax.rem(c, 4)
      b2 = lax.rem(c + 2, 4)
      j = lax.rem(c, 6)
      j2 = lax.rem(c + 2, 6)
      j3 = lax.rem(c + 3, 6)

      @pl.when(c + 2 < n_my)
      def _():
        @pl.when(c >= 2)
        def _():
          pltpu.make_async_copy(rows_v.at[b2], acc_s.at[idx_v.at[j2, 1]],
                                ssem.at[b2]).wait()
        @pl.when(c >= 1)
        def _():
          pltpu.make_async_copy(edges_hbm.at[start + c + 2], idx_v.at[j2],
                                isem.at[j2]).wait()
        pltpu.async_copy(h_hbm.at[idx_v.at[j2, 0]], rows_v.at[b2],
                         gsem.at[b2])

      @pl.when(c + 3 < n_my)
      def _():
        pltpu.async_copy(edges_hbm.at[start + c + 3], idx_v.at[j3],
                         isem.at[j3])

      pltpu.make_async_copy(h_hbm.at[idx_v.at[j, 0]], rows_v.at[b],
                            gsem.at[b]).wait()
      pltpu.async_copy(rows_v.at[b], acc_s.at[idx_v.at[j, 1]], ssem.at[b],
                       add=True)
      return carry

    lax.fori_loop(0, n_my, body, 0)

    def drain(kk, carry):
      pltpu.make_async_copy(rows_v.at[lax.rem(kk, 4)],
                            acc_s.at[idx_v.at[lax.rem(kk, 6), 1]],
                            ssem.at[lax.rem(kk, 4)]).wait()
      return carry

    lax.fori_loop(n_my - 4, n_my, drain, 0)
    plsc.subcore_barrier()

    def wbody(j, carry):
      r = sid * RPT + j * CHUNK
      pltpu.sync_copy(acc_s.at[pl.ds(r, CHUNK)], rows_v.at[0])
      pltpu.sync_copy(rows_v.at[0], out_hbm.at[cid, pl.ds(r, CHUNK)])
      return carry

    lax.fori_loop(0, RPT // CHUNK, wbody, 0)
    _tail = RPT - (RPT // CHUNK) * CHUNK
    r = sid * RPT + (RPT // CHUNK) * CHUNK
    pltpu.sync_copy(acc_s.at[pl.ds(r, _tail)], rows_v.at[0, pl.ds(0, _tail)])
    pltpu.sync_copy(rows_v.at[0, pl.ds(0, _tail)],
                    out_hbm.at[cid, pl.ds(r, _tail)])

  return k


_R = 1264


def _tc1(p, w, b):
  def body(p_ref, w_ref, b_ref, h_ref, rc_ref):
    s = p_ref[0] + p_ref[1]
    rc = 1.0 / jnp.maximum(s[:, 128:129], 1.0)
    x = s[:, :128] * rc
    y = lax.dot_general(x, w_ref[...], (((1,), (1,)), ((), ())),
                        preferred_element_type=jnp.float32) + b_ref[...]
    h_ref[...] = jnp.where(y > 0, y, jnp.exp(jnp.minimum(y, 0.0)) - 1.0)
    rc_ref[...] = rc

  return pl.pallas_call(
      body,
      grid=(NP // _R,),
      in_specs=[
          pl.BlockSpec((2, _R, 144), lambda i: (0, i, 0)),
          pl.BlockSpec((128, 128), lambda i: (0, 0)),
          pl.BlockSpec((1, 128), lambda i: (0, 0)),
      ],
      out_specs=[
          pl.BlockSpec((_R, 128), lambda i: (i, 0)),
          pl.BlockSpec((_R, 1), lambda i: (i, 0)),
      ],
      out_shape=[
          jax.ShapeDtypeStruct((NP, 128), jnp.float32),
          jax.ShapeDtypeStruct((NP, 1), jnp.float32),
      ],
  )(p, w, b)


def _tc2(p, rc, w, b):
  def body(p_ref, rc_ref, w_ref, b_ref, o_ref):
    x = (p_ref[0] + p_ref[1]) * rc_ref[...]
    y = lax.dot_general(x, w_ref[...], (((1,), (1,)), ((), ())),
                        preferred_element_type=jnp.float32) + b_ref[...]
    o_ref[...] = jnp.where(y > 0, y, jnp.exp(jnp.minimum(y, 0.0)) - 1.0)

  return pl.pallas_call(
      body,
      grid=(NP // _R,),
      in_specs=[
          pl.BlockSpec((2, _R, 128), lambda i: (0, i, 0)),
          pl.BlockSpec((_R, 1), lambda i: (i, 0)),
          pl.BlockSpec((128, 128), lambda i: (0, 0)),
          pl.BlockSpec((1, 128), lambda i: (0, 0)),
      ],
      out_specs=pl.BlockSpec((_R, 128), lambda i: (i, 0)),
      out_shape=jax.ShapeDtypeStruct((NP, 128), jnp.float32),
  )(p, rc, w, b)


def kernel(h, edge_index, W1, b1, W2, b2):
  E = edge_index.shape[1]
  pad = EPAD - E
  src = jnp.concatenate([edge_index[0].astype(jnp.int32),
                         jnp.zeros((pad,), jnp.int32)]).reshape(TOTCH, CHUNK)
  dst = jnp.concatenate([edge_index[1].astype(jnp.int32),
                         jnp.full((pad,), N_NODES, jnp.int32)]
                        ).reshape(TOTCH, CHUNK)
  edges = jnp.stack([src, dst], axis=1)
  h_aug = jnp.concatenate(
      [h, jnp.ones((N_NODES, 1), jnp.float32),
       jnp.zeros((N_NODES, 15), jnp.float32)], axis=1)

  z144 = jnp.zeros((CHUNK, 144), jnp.float32)
  z128 = jnp.zeros((CHUNK, 128), jnp.float32)

  p1 = _sc_agg(144)(h_aug, edges, z144)
  h1, rc = _tc1(p1, W1, b1.reshape(1, 128))
  p2 = _sc_agg(128)(h1, edges, z128)
  out = _tc2(p2, rc, W2, b2.reshape(1, 128))
  return out[:N_NODES]
